# Initial kernel scaffold; baseline (speedup 1.0000x reference)
import jax
import jax.numpy as jnp
from jax.experimental import pallas as pl


def kernel(features, edge_index, adj_vals, weight):
    raise NotImplementedError("write your pallas kernel here")


# speedup vs baseline: 2.9254x; 2.9254x over previous
import functools

import jax
import jax.numpy as jnp
from jax import lax
from jax.experimental import pallas as pl
from jax.experimental.pallas import tpu as pltpu
from jax.experimental.pallas import tpu_sc as plsc

N_NODES = 10000
N_EDGES = 320000
FEAT = 128

NUM_CORES = 2
NUM_SUBCORES = 16
NUM_WORKERS = NUM_CORES * NUM_SUBCORES
CHUNK = 128
CHUNKS_PER_WORKER = 80
E_PAD = NUM_WORKERS * CHUNKS_PER_WORKER * CHUNK
N_PAD = 10240
ROWS_PER_TILE = N_PAD // NUM_SUBCORES


def _sc_aggregate(features, rows2d, cols2d, vals2d, zeros):
  mesh = plsc.VectorSubcoreMesh(core_axis_name="c", subcore_axis_name="s")

  @functools.partial(
      pl.kernel,
      out_type=jax.ShapeDtypeStruct((NUM_CORES, N_PAD, FEAT), jnp.float32),
      mesh=mesh,
      scratch_types=[
          pltpu.VMEM((CHUNKS_PER_WORKER, CHUNK), jnp.int32),
          pltpu.VMEM((CHUNKS_PER_WORKER, CHUNK), jnp.int32),
          pltpu.VMEM((CHUNKS_PER_WORKER, CHUNK), jnp.float32),
          pltpu.VMEM((CHUNK, FEAT), jnp.float32),
          pltpu.VMEM_SHARED((N_PAD, FEAT), jnp.float32),
      ],
  )
  def k(feat_hbm, rows_hbm, cols_hbm, vals_hbm, zeros_hbm, out_hbm,
        cols_v, rows_v, vals_v, buf_v, acc_s):
    c = lax.axis_index("c")
    s = lax.axis_index("s")
    wid = c * NUM_SUBCORES + s
    base = wid * CHUNKS_PER_WORKER

    pltpu.sync_copy(cols_hbm.at[pl.ds(base, CHUNKS_PER_WORKER)], cols_v)
    pltpu.sync_copy(rows_hbm.at[pl.ds(base, CHUNKS_PER_WORKER)], rows_v)
    pltpu.sync_copy(vals_hbm.at[pl.ds(base, CHUNKS_PER_WORKER)], vals_v)
    pltpu.sync_copy(zeros_hbm, acc_s.at[pl.ds(s * ROWS_PER_TILE, ROWS_PER_TILE)])
    plsc.subcore_barrier()

    def chunk_body(j, carry):
      pltpu.sync_copy(feat_hbm.at[cols_v.at[j]], buf_v)

      def grp(g, carry2):
        vv = vals_v[j, pl.ds(g * 16, 16)]
        for e in range(16):
          vb = jnp.take(vv, jnp.full((16,), e, jnp.int32))
          r = g * 16 + e
          for f in range(FEAT // 16):
            buf_v[r, pl.ds(f * 16, 16)] = buf_v[r, pl.ds(f * 16, 16)] * vb
        return carry2

      lax.fori_loop(0, CHUNK // 16, grp, 0)

      pltpu.sync_copy(buf_v, acc_s.at[rows_v.at[j]], add=True)
      return carry

    lax.fori_loop(0, CHUNKS_PER_WORKER, chunk_body, 0)
    plsc.subcore_barrier()

    pltpu.sync_copy(
        acc_s.at[pl.ds(s * ROWS_PER_TILE, ROWS_PER_TILE)],
        out_hbm.at[c].at[pl.ds(s * ROWS_PER_TILE, ROWS_PER_TILE)])

  return k(features, rows2d, cols2d, vals2d, zeros)


def _tc_finish(parts, weight):
  blk = 1024

  def body(p_ref, w_ref, o_ref):
    s = p_ref[0] + p_ref[1]
    o_ref[:] = jnp.maximum(
        jnp.dot(s, w_ref[:], preferred_element_type=jnp.float32), 0.0)

  return pl.pallas_call(
      body,
      grid=(N_PAD // blk,),
      in_specs=[
          pl.BlockSpec((NUM_CORES, blk, FEAT), lambda i: (0, i, 0)),
          pl.BlockSpec((FEAT, FEAT), lambda i: (0, 0)),
      ],
      out_specs=pl.BlockSpec((blk, FEAT), lambda i: (i, 0)),
      out_shape=jax.ShapeDtypeStruct((N_PAD, FEAT), jnp.float32),
  )(parts, weight)


def kernel(features, edge_index, adj_vals, weight):
  rows = edge_index[0].astype(jnp.int32)
  cols = edge_index[1].astype(jnp.int32)
  pad = E_PAD - N_EDGES
  rows2 = jnp.concatenate([rows, jnp.zeros((pad,), jnp.int32)])
  cols2 = jnp.concatenate([cols, jnp.zeros((pad,), jnp.int32)])
  vals2 = jnp.concatenate([adj_vals.astype(jnp.float32),
                           jnp.zeros((pad,), jnp.float32)])
  rows2 = rows2.reshape(E_PAD // CHUNK, CHUNK)
  cols2 = cols2.reshape(E_PAD // CHUNK, CHUNK)
  vals2 = vals2.reshape(E_PAD // CHUNK, CHUNK)
  zeros = jnp.zeros((ROWS_PER_TILE, FEAT), jnp.float32)

  parts = _sc_aggregate(features, rows2, cols2, vals2, zeros)
  return _tc_finish(parts, weight)[:N_NODES]
